# Initial kernel scaffold; baseline (speedup 1.0000x reference)
import jax
import jax.numpy as jnp
from jax.experimental import pallas as pl


def kernel(text, emb_table, W, b):
    raise NotImplementedError("write your pallas kernel here")


# speedup vs baseline: 3.5933x; 3.5933x over previous
import functools

import jax
import jax.numpy as jnp
from jax import lax
from jax.experimental import pallas as pl
from jax.experimental.pallas import tpu as pltpu
from jax.experimental.pallas import tpu_sc as plsc

VOCAB = 100000
EMB = 128
OUT = 256
NTOK = 1024 * 200

NC = 2
NS = 16
NW = NC * NS
BPW = NTOK // NW
CHUNK = 128
NCHUNK = BPW // CHUNK


def _gather_body(tab_hbm, idx_hbm, out_hbm, idx_v, buf_v, gsem):
    wid = lax.axis_index("s") * NC + lax.axis_index("c")
    base = wid * BPW
    pltpu.sync_copy(idx_hbm.at[pl.ds(base, BPW)], idx_v)

    def chunk(j, carry):
        off = j * CHUNK
        pltpu.async_copy(
            tab_hbm.at[idx_v.at[pl.ds(off, CHUNK)]], buf_v, gsem
        ).wait()
        pltpu.sync_copy(buf_v, out_hbm.at[pl.ds(base + off, CHUNK)])
        return carry

    lax.fori_loop(0, NCHUNK, chunk, 0)


def _sc_gather(emb_table, idx):
    mesh = plsc.VectorSubcoreMesh(core_axis_name="c", subcore_axis_name="s")
    f = pl.kernel(
        _gather_body,
        out_type=jax.ShapeDtypeStruct((NTOK, EMB), jnp.float32),
        mesh=mesh,
        scratch_types=[
            pltpu.VMEM((BPW,), jnp.int32),
            pltpu.VMEM((CHUNK, EMB), jnp.float32),
            pltpu.SemaphoreType.DMA,
        ],
    )
    return f(emb_table, idx)


RB = 2048


def _mm_body(x_ref, w_ref, b_ref, o_ref):
    o_ref[...] = jnp.maximum(
        jnp.dot(x_ref[...], w_ref[...], preferred_element_type=jnp.float32)
        + b_ref[...],
        0.0,
    )


def _tc_matmul(x, W, b2d):
    return pl.pallas_call(
        _mm_body,
        grid=(NTOK // RB,),
        in_specs=[
            pl.BlockSpec((RB, EMB), lambda i: (i, 0)),
            pl.BlockSpec((EMB, OUT), lambda i: (0, 0)),
            pl.BlockSpec((1, OUT), lambda i: (0, 0)),
        ],
        out_specs=pl.BlockSpec((RB, OUT), lambda i: (i, 0)),
        out_shape=jax.ShapeDtypeStruct((NTOK, OUT), jnp.float32),
    )(x, W, b2d)


def kernel(text, emb_table, W, b):
    B, L = text.shape
    idx = text.reshape(-1).astype(jnp.int32)
    x = _sc_gather(emb_table, idx)
    y = _tc_matmul(x, W, b.reshape(1, OUT))
    return y.reshape(B, L, OUT)


# speedup vs baseline: 4.0609x; 1.1301x over previous
import functools

import jax
import jax.numpy as jnp
from jax import lax
from jax.experimental import pallas as pl
from jax.experimental.pallas import tpu as pltpu
from jax.experimental.pallas import tpu_sc as plsc

VOCAB = 100000
EMB = 128
OUT = 256
NTOK = 1024 * 200

NC = 2
NS = 16
NW = NC * NS
BPW = NTOK // NW
CHUNK = 128
NCHUNK = BPW // CHUNK


def _gather_body(tab_hbm, idx_hbm, out_hbm, idx_v, buf_v, gsem):
    wid = lax.axis_index("s") * NC + lax.axis_index("c")
    base = wid * BPW
    pltpu.sync_copy(idx_hbm.at[pl.ds(base, BPW)], idx_v)

    def chunk(j, carry):
        off = j * CHUNK
        pltpu.async_copy(
            tab_hbm.at[idx_v.at[pl.ds(off, CHUNK)]], buf_v, gsem
        ).wait()
        pltpu.sync_copy(buf_v, out_hbm.at[pl.ds(base + off, CHUNK)])
        return carry

    lax.fori_loop(0, NCHUNK, chunk, 0)


def _sc_gather(emb_table, idx):
    mesh = plsc.VectorSubcoreMesh(core_axis_name="c", subcore_axis_name="s")
    f = pl.kernel(
        _gather_body,
        out_type=jax.ShapeDtypeStruct((NTOK, EMB), jnp.float32),
        mesh=mesh,
        scratch_types=[
            pltpu.VMEM((BPW,), jnp.int32),
            pltpu.VMEM((CHUNK, EMB), jnp.float32),
            pltpu.SemaphoreType.DMA,
        ],
    )
    return f(emb_table, idx)


RB = 4096


def _mm_body(x_ref, w_ref, b_ref, o_ref):
    o_ref[...] = jnp.maximum(
        jnp.dot(x_ref[...], w_ref[...], preferred_element_type=jnp.float32)
        + b_ref[...],
        0.0,
    )


def _tc_matmul(x, W, b2d):
    return pl.pallas_call(
        _mm_body,
        grid=(NTOK // RB,),
        in_specs=[
            pl.BlockSpec((RB, EMB), lambda i: (i, 0)),
            pl.BlockSpec((EMB, OUT), lambda i: (0, 0)),
            pl.BlockSpec((1, OUT), lambda i: (0, 0)),
        ],
        out_specs=pl.BlockSpec((RB, OUT), lambda i: (i, 0)),
        out_shape=jax.ShapeDtypeStruct((NTOK, OUT), jnp.float32),
    )(x, W, b2d)


def kernel(text, emb_table, W, b):
    B, L = text.shape
    idx = text.reshape(-1).astype(jnp.int32)
    x = _sc_gather(emb_table, idx)
    y = _tc_matmul(x, W, b.reshape(1, OUT))
    return y.reshape(B, L, OUT)


# speedup vs baseline: 4.1414x; 1.0198x over previous
import functools

import jax
import jax.numpy as jnp
from jax import lax
from jax.experimental import pallas as pl
from jax.experimental.pallas import tpu as pltpu
from jax.experimental.pallas import tpu_sc as plsc

VOCAB = 100000
EMB = 128
OUT = 256
NTOK = 1024 * 200

NC = 2
NS = 16
NW = NC * NS
BPW = NTOK // NW
CHUNK = 128
NCHUNK = BPW // CHUNK


def _gather_body(tab_hbm, idx_hbm, out_hbm, idx_v, buf_v, gsem):
    wid = lax.axis_index("s") * NC + lax.axis_index("c")
    base = wid * BPW
    pltpu.sync_copy(idx_hbm.at[pl.ds(base, BPW)], idx_v)

    def chunk(j, carry):
        off = j * CHUNK
        pltpu.async_copy(
            tab_hbm.at[idx_v.at[pl.ds(off, CHUNK)]], buf_v, gsem
        ).wait()
        pltpu.sync_copy(buf_v, out_hbm.at[pl.ds(base + off, CHUNK)])
        return carry

    lax.fori_loop(0, NCHUNK, chunk, 0)


def _sc_gather(emb_table, idx):
    mesh = plsc.VectorSubcoreMesh(core_axis_name="c", subcore_axis_name="s")
    f = pl.kernel(
        _gather_body,
        out_type=jax.ShapeDtypeStruct((NTOK, EMB), jnp.float32),
        mesh=mesh,
        scratch_types=[
            pltpu.VMEM((BPW,), jnp.int32),
            pltpu.VMEM((CHUNK, EMB), jnp.float32),
            pltpu.SemaphoreType.DMA,
        ],
    )
    return f(emb_table, idx)


RB = 8192


def _mm_body(x_ref, w_ref, b_ref, o_ref):
    o_ref[...] = jnp.maximum(
        jnp.dot(x_ref[...], w_ref[...], preferred_element_type=jnp.float32)
        + b_ref[...],
        0.0,
    )


def _tc_matmul(x, W, b2d):
    return pl.pallas_call(
        _mm_body,
        grid=(NTOK // RB,),
        in_specs=[
            pl.BlockSpec((RB, EMB), lambda i: (i, 0)),
            pl.BlockSpec((EMB, OUT), lambda i: (0, 0)),
            pl.BlockSpec((1, OUT), lambda i: (0, 0)),
        ],
        out_specs=pl.BlockSpec((RB, OUT), lambda i: (i, 0)),
        out_shape=jax.ShapeDtypeStruct((NTOK, OUT), jnp.float32),
    )(x, W, b2d)


def kernel(text, emb_table, W, b):
    B, L = text.shape
    idx = text.reshape(-1).astype(jnp.int32)
    x = _sc_gather(emb_table, idx)
    y = _tc_matmul(x, W, b.reshape(1, OUT))
    return y.reshape(B, L, OUT)
